# baseline (speedup 1.0000x reference)
import functools

import jax
import jax.numpy as jnp
from jax import lax
from jax.experimental import pallas as pl
from jax.experimental.pallas import tpu as pltpu
from jax.experimental.pallas import tpu_sc as plsc

_TOTAL = 16384
_D = 512
_NB = 16
_BLK = 4096
_NW = 16
_CHUNK = _TOTAL // _NW
_NV = _CHUNK // 16


def _tc_body(x_ref, w_ref, benc_ref, waggt_ref, r_ref):
    h = jnp.maximum(
        jnp.dot(x_ref[...], w_ref[...], preferred_element_type=jnp.float32)
        + benc_ref[...], 0.0)
    r_ref[...] = jnp.sum(h * waggt_ref[...], axis=1, keepdims=True)


def _sc_body(r_hbm, starts_hbm, ends_hbm, bagg_hbm, out_hbm,
             r_v, st_v, en_v, bg_v, acc_v, all_v, out_v, shared, sem):
    wid = lax.axis_index("s")
    base = wid * _CHUNK
    pltpu.sync_copy(r_hbm.at[pl.ds(base, _CHUNK)], r_v)
    pltpu.sync_copy(starts_hbm, st_v)
    pltpu.sync_copy(ends_hbm, en_v)

    lane = lax.iota(jnp.int32, 16)
    starts_vec = st_v[...]
    ends_vec = en_v[...]
    sb = [starts_vec[b] for b in range(_NB)]
    eb = [ends_vec[b] for b in range(_NB)]

    def body(v, accs):
        rv = r_v[pl.ds(v * 16, 16)]
        g = base + v * 16 + lane
        return tuple(
            accs[b] + jnp.where((g >= sb[b]) & (g < eb[b]), rv, 0.0)
            for b in range(_NB))

    accs = lax.fori_loop(
        0, _NV, body, tuple(jnp.zeros((16,), jnp.float32)
                            for _ in range(_NB)))

    out = jnp.zeros((16,), jnp.float32)
    for b in range(_NB):
        tot_b = accs[b][0]
        for l in range(1, 16):
            tot_b = tot_b + accs[b][l]
        out = jnp.where(lane == b, tot_b, out)
    acc_v[...] = out

    pltpu.sync_copy(acc_v, shared.at[pl.ds(wid * 16, 16)])
    plsc.subcore_barrier()

    @pl.when(wid == 0)
    def _fin():
        pltpu.sync_copy(shared, all_v)
        pltpu.sync_copy(bagg_hbm, bg_v)
        tot = jnp.zeros((16,), jnp.float32)
        for w in range(_NW):
            tot = tot + all_v[pl.ds(w * 16, 16)]
        counts = jnp.maximum((en_v[...] - st_v[...]).astype(jnp.float32), 1.0)
        out_v[...] = tot / counts + bg_v[...]
        pltpu.sync_copy(out_v, out_hbm)


def kernel(x, bag_sizes, W_enc, b_enc, W_agg, b_agg):
    starts = bag_sizes[:_NB]
    ends = bag_sizes[1:]
    waggt = W_agg.reshape(1, _D)
    benc = b_enc.reshape(1, _D)
    bagg16 = jnp.broadcast_to(b_agg, (_NB,))

    grid = _TOTAL // _BLK
    r = pl.pallas_call(
        _tc_body,
        grid=(grid,),
        in_specs=[
            pl.BlockSpec((_BLK, _D), lambda i: (i, 0)),
            pl.BlockSpec((_D, _D), lambda i: (0, 0)),
            pl.BlockSpec((1, _D), lambda i: (0, 0)),
            pl.BlockSpec((1, _D), lambda i: (0, 0)),
        ],
        out_specs=pl.BlockSpec((_BLK, 1), lambda i: (i, 0)),
        out_shape=jax.ShapeDtypeStruct((_TOTAL, 1), jnp.float32),
        compiler_params=pltpu.CompilerParams(
            dimension_semantics=("arbitrary",)),
    )(x, W_enc, benc, waggt)

    mesh = plsc.VectorSubcoreMesh(core_axis_name="c", subcore_axis_name="s",
                                  num_cores=1)
    f = pl.kernel(
        _sc_body,
        out_type=jax.ShapeDtypeStruct((_NB,), jnp.float32),
        mesh=mesh,
        scratch_types=[
            pltpu.VMEM((_CHUNK,), jnp.float32),
            pltpu.VMEM((16,), jnp.int32),
            pltpu.VMEM((16,), jnp.int32),
            pltpu.VMEM((16,), jnp.float32),
            pltpu.VMEM((16,), jnp.float32),
            pltpu.VMEM((_NW * 16,), jnp.float32),
            pltpu.VMEM((16,), jnp.float32),
            pltpu.VMEM_SHARED((_NW * 16,), jnp.float32),
            pltpu.SemaphoreType.DMA,
        ],
    )
    out = f(r.reshape(_TOTAL), starts, ends, bagg16)
    return out.reshape(_NB, 1)
